# baseline (speedup 1.0000x reference)
import functools

import jax
import jax.numpy as jnp
from jax import lax
from jax.experimental import pallas as pl
from jax.experimental.pallas import tpu as pltpu
from jax.experimental.pallas import tpu_sc as plsc

N = 10000
E = 320000
D = 128
H = 128
O = 10
G = 64

NC = 2
NS = 16
NW = NC * NS

K = 128
CPW = 80
CH_TOT = NW * CPW
E_PAD = CH_TOT * K

NP = 10112
RPT = NP // NS

_sc_mesh = plsc.VectorSubcoreMesh(
    core_axis_name="c", subcore_axis_name="s", num_cores=NC, num_subcores=NS)


@functools.partial(
    pl.kernel,
    out_type=jax.ShapeDtypeStruct((NC, NP, H // 2), jnp.float32),
    mesh=_sc_mesh,
    compiler_params=pltpu.CompilerParams(use_tc_tiling_on_sc=False),
    scratch_types=[
        pltpu.VMEM((CPW, K), jnp.int32),
        pltpu.VMEM((K, H // 2), jnp.float32),
        pltpu.VMEM_SHARED((NP, H // 2), jnp.float32),
        pltpu.SemaphoreType.DMA,
    ],
)
def _sc_degree(dst_hbm, zeros_hbm, ones_hbm, out_hbm, dst_v, ones_v, acc,
               sem):
    c = lax.axis_index("c")
    s = lax.axis_index("s")
    w = c * NS + s
    pltpu.sync_copy(zeros_hbm.at[pl.ds(s * RPT, RPT)],
                    acc.at[pl.ds(s * RPT, RPT)])
    pltpu.sync_copy(ones_hbm, ones_v)
    pltpu.sync_copy(dst_hbm.at[pl.ds(w * CPW, CPW)], dst_v)
    plsc.subcore_barrier()

    W = 8

    def issue(j, _):
        pltpu.async_copy(ones_v, acc.at[dst_v.at[j]], sem, add=True)
        return _

    def issue_drain(j, _):
        pltpu.async_copy(ones_v, acc.at[dst_v.at[j]], sem, add=True)
        pltpu.make_async_copy(ones_v, acc.at[dst_v.at[j]], sem).wait()
        return _

    lax.fori_loop(0, W, issue, None)
    lax.fori_loop(W, CPW, issue_drain, None)
    for _ in range(W):
        pltpu.make_async_copy(ones_v, acc.at[dst_v.at[0]], sem).wait()
    plsc.subcore_barrier()
    pltpu.sync_copy(acc.at[pl.ds(s * RPT, RPT)],
                    out_hbm.at[c].at[pl.ds(s * RPT, RPT)])


HH = H // 2
CPT = CH_TOT // NS
NBUF = 4
NPH = 4
CPP = CPT // NPH
ZRT = N // NS


@functools.partial(
    pl.kernel,
    out_type=jax.ShapeDtypeStruct((NC, NP, HH), jnp.float32),
    mesh=_sc_mesh,
    compiler_params=pltpu.CompilerParams(use_tc_tiling_on_sc=False),
    scratch_types=[
        pltpu.VMEM((CPP, K), jnp.int32),
        pltpu.VMEM((CPP, K), jnp.int32),
    ] + [pltpu.VMEM((K, HH), jnp.float32) for _ in range(NBUF)] + [
        pltpu.VMEM_SHARED((N, HH), jnp.float32),
        pltpu.VMEM_SHARED((NP, HH), jnp.float32),
    ] + [pltpu.SemaphoreType.DMA for _ in range(2 * NBUF)],
)
def _sc_scatter(src_hbm, dst_hbm, z_hbm, zeros_hbm, out_hbm,
                src_v, dst_v, *rest):
    bufs = rest[:NBUF]
    zsp = rest[NBUF]
    acc = rest[NBUF + 1]
    gsems = rest[NBUF + 2:NBUF + 2 + NBUF]
    ssems = rest[NBUF + 2 + NBUF:]
    c = lax.axis_index("c")
    s = lax.axis_index("s")
    pltpu.sync_copy(zeros_hbm.at[pl.ds(s * RPT, RPT)],
                    acc.at[pl.ds(s * RPT, RPT)])
    pltpu.sync_copy(z_hbm.at[c].at[pl.ds(s * ZRT, ZRT)],
                    zsp.at[pl.ds(s * ZRT, ZRT)])
    plsc.subcore_barrier()

    def wg(b, j):
        pltpu.make_async_copy(zsp.at[src_v.at[j]], bufs[b], gsems[b]).wait()

    def ss(b, j):
        pltpu.async_copy(bufs[b], acc.at[dst_v.at[j]], ssems[b], add=True)

    def ws(b, j):
        pltpu.make_async_copy(bufs[b], acc.at[dst_v.at[j]], ssems[b]).wait()

    for p in range(NPH):
        base = s * CPT + p * CPP
        pltpu.sync_copy(src_hbm.at[pl.ds(base, CPP)], src_v)
        pltpu.sync_copy(dst_hbm.at[pl.ds(base, CPP)], dst_v)

        for b in range(NBUF):
            pltpu.async_copy(zsp.at[src_v.at[b]], bufs[b], gsems[b])
        wg(0, 0)
        ss(0, 0)

        def body(t, _):
            jb = NBUF * t + 1
            for b in range(NBUF):
                j = jb + b
                wg((b + 1) % NBUF, j)
                ss((b + 1) % NBUF, j)
                ws(b, j - 1)
                pltpu.async_copy(zsp.at[src_v.at[j + NBUF - 1]],
                                 bufs[b], gsems[b])
            return _

        lax.fori_loop(0, (CPP - NBUF) // NBUF, body, None)
        for j in range(CPP - NBUF + 1, CPP):
            wg(j % NBUF, j)
            ss(j % NBUF, j)
            ws((j - 1) % NBUF, j - 1)
        ws((CPP - 1) % NBUF, CPP - 1)

    plsc.subcore_barrier()
    pltpu.sync_copy(acc.at[pl.ds(s * RPT, RPT)],
                    out_hbm.at[c].at[pl.ds(s * RPT, RPT)])


def _halves(ref):
    return jnp.concatenate([ref[0, :N, :], ref[1, :N, :]], axis=1)


def _split(ref, val):
    ref[0, :, :] = val[:, :HH]
    ref[1, :, :] = val[:, HH:]


def _tc_mm1(x_ref, w1_ref, u1_ref):
    u1_ref[...] = jnp.dot(x_ref[...], w1_ref[...],
                          preferred_element_type=jnp.float32)


def _tc_first(deg2_ref, u1_ref, z1_ref, dis_ref):
    degh = deg2_ref[0] + deg2_ref[1] + 1.0
    dish = lax.rsqrt(degh)
    dis = jnp.concatenate([dish, dish], axis=1)
    dis_ref[...] = dis
    _split(z1_ref, u1_ref[...] * dis)


def _tc_mid(dis_ref, s1_ref, z1_ref, b1_ref, w2_ref, z2_ref):
    dis = dis_ref[...]
    agg = (jnp.concatenate([s1_ref[0], s1_ref[1]], axis=1)
           + jnp.concatenate([z1_ref[0], z1_ref[1]], axis=1))
    h1 = jnp.maximum(agg * dis + b1_ref[...], 0.0)
    _split(z2_ref, jnp.dot(h1, w2_ref[...],
                           preferred_element_type=jnp.float32) * dis)


def _tc_last(dis_ref, s2_ref, z2_ref, b2_ref, batch_ref, wlin_ref, blin_ref,
             out_ref):
    dis = dis_ref[...]
    agg = _halves(s2_ref) + _halves(z2_ref)
    h2 = jnp.maximum(agg * dis + b2_ref[...], 0.0)
    gid = lax.broadcasted_iota(jnp.int32, (G, N), 0)
    onehot = (batch_ref[...] == gid).astype(jnp.float32)
    sums = jnp.dot(onehot, h2, preferred_element_type=jnp.float32)
    cnt = jnp.sum(onehot, axis=1, keepdims=True)
    g = sums / jnp.maximum(cnt, 1.0)
    out_ref[...] = jnp.dot(g, wlin_ref[...],
                           preferred_element_type=jnp.float32) + blin_ref[...]


def _tc_call(body, out_shape, *args):
    return pl.pallas_call(body, out_shape=out_shape)(*args)


BM = 1000


def _tc_first_call(deg2, u1):
    bs_h2 = pl.BlockSpec((NC, BM, HH), lambda i: (0, i, 0))
    return pl.pallas_call(
        _tc_first,
        grid=(N // BM,),
        in_specs=[bs_h2, pl.BlockSpec((BM, H), lambda i: (i, 0))],
        out_specs=(bs_h2, pl.BlockSpec((BM, H), lambda i: (i, 0))),
        out_shape=(jax.ShapeDtypeStruct((NC, N, HH), jnp.float32),
                   jax.ShapeDtypeStruct((N, H), jnp.float32)),
    )(deg2, u1)


def _tc_mid_call(dis, s1, z1, b1, W2):
    bs_h2 = pl.BlockSpec((NC, BM, HH), lambda i: (0, i, 0))
    return pl.pallas_call(
        _tc_mid,
        grid=(N // BM,),
        in_specs=[
            pl.BlockSpec((BM, H), lambda i: (i, 0)),
            bs_h2,
            bs_h2,
            pl.BlockSpec((1, H), lambda i: (0, 0)),
            pl.BlockSpec((H, H), lambda i: (0, 0)),
        ],
        out_specs=bs_h2,
        out_shape=jax.ShapeDtypeStruct((NC, N, HH), jnp.float32),
    )(dis, s1, z1, b1, W2)


def kernel(x, edge_index, batch, W1, b1, W2, b2, Wlin, blin):
    src = edge_index[0]
    dst = edge_index[1]
    pad = E_PAD - E
    srcp = jnp.concatenate([src, jnp.zeros((pad,), jnp.int32)]).reshape(CH_TOT, K)
    dstp = jnp.concatenate([dst, jnp.full((pad,), N, jnp.int32)]).reshape(CH_TOT, K)

    zeros_h_hbm = jnp.zeros((NP, HH), jnp.float32)
    ones_hbm = jnp.ones((K, HH), jnp.float32)

    deg2 = _sc_degree(dstp, zeros_h_hbm, ones_hbm)
    u1 = _tc_call(_tc_mm1, jax.ShapeDtypeStruct((N, H), jnp.float32), x, W1)

    z1, dis = _tc_first_call(deg2, u1)
    s1 = _sc_scatter(srcp, dstp, z1, zeros_h_hbm)
    z2 = _tc_mid_call(dis, s1, z1, b1.reshape(1, H), W2)
    s2 = _sc_scatter(srcp, dstp, z2, zeros_h_hbm)
    out = _tc_call(_tc_last, jax.ShapeDtypeStruct((G, O), jnp.float32),
                   dis, s2, z2, b2.reshape(1, H), batch.reshape(1, N),
                   Wlin, blin.reshape(1, O))
    return out


# speedup vs baseline: 1.0069x; 1.0069x over previous
import functools

import jax
import jax.numpy as jnp
from jax import lax
from jax.experimental import pallas as pl
from jax.experimental.pallas import tpu as pltpu
from jax.experimental.pallas import tpu_sc as plsc

N = 10000
E = 320000
D = 128
H = 128
O = 10
G = 64

NC = 2
NS = 16
NW = NC * NS

K = 128
CPW = 80
CH_TOT = NW * CPW
E_PAD = CH_TOT * K

NP = 10112
RPT = NP // NS

_sc_mesh = plsc.VectorSubcoreMesh(
    core_axis_name="c", subcore_axis_name="s", num_cores=NC, num_subcores=NS)


@functools.partial(
    pl.kernel,
    out_type=jax.ShapeDtypeStruct((NC, NP, H // 2), jnp.float32),
    mesh=_sc_mesh,
    compiler_params=pltpu.CompilerParams(use_tc_tiling_on_sc=False),
    scratch_types=[
        pltpu.VMEM((CPW, K), jnp.int32),
        pltpu.VMEM((K, H // 2), jnp.float32),
        pltpu.VMEM_SHARED((NP, H // 2), jnp.float32),
        pltpu.SemaphoreType.DMA,
    ],
)
def _sc_degree(dst_hbm, zeros_hbm, ones_hbm, out_hbm, dst_v, ones_v, acc,
               sem):
    c = lax.axis_index("c")
    s = lax.axis_index("s")
    w = c * NS + s
    pltpu.sync_copy(zeros_hbm.at[pl.ds(s * RPT, RPT)],
                    acc.at[pl.ds(s * RPT, RPT)])
    pltpu.sync_copy(ones_hbm, ones_v)
    pltpu.sync_copy(dst_hbm.at[pl.ds(w * CPW, CPW)], dst_v)
    plsc.subcore_barrier()

    W = 8

    def issue(j, _):
        pltpu.async_copy(ones_v, acc.at[dst_v.at[j]], sem, add=True)
        return _

    def issue_drain(j, _):
        pltpu.async_copy(ones_v, acc.at[dst_v.at[j]], sem, add=True)
        pltpu.make_async_copy(ones_v, acc.at[dst_v.at[j]], sem).wait()
        return _

    lax.fori_loop(0, W, issue, None)
    lax.fori_loop(W, CPW, issue_drain, None)
    for _ in range(W):
        pltpu.make_async_copy(ones_v, acc.at[dst_v.at[0]], sem).wait()
    plsc.subcore_barrier()
    pltpu.sync_copy(acc.at[pl.ds(s * RPT, RPT)],
                    out_hbm.at[c].at[pl.ds(s * RPT, RPT)])


HH = H // 2
CPT = CH_TOT // NS
NBUF = 4
NPH = 4
CPP = CPT // NPH
ZRT = N // NS


@functools.partial(
    pl.kernel,
    out_type=jax.ShapeDtypeStruct((NC, NP, HH), jnp.float32),
    mesh=_sc_mesh,
    compiler_params=pltpu.CompilerParams(use_tc_tiling_on_sc=False),
    scratch_types=[
        pltpu.VMEM((CPP, K), jnp.int32),
        pltpu.VMEM((CPP, K), jnp.int32),
    ] + [pltpu.VMEM((K, HH), jnp.float32) for _ in range(NBUF)] + [
        pltpu.VMEM_SHARED((N, HH), jnp.float32),
        pltpu.VMEM_SHARED((NP, HH), jnp.float32),
    ] + [pltpu.SemaphoreType.DMA for _ in range(2 * NBUF)],
)
def _sc_scatter(src_hbm, dst_hbm, z_hbm, zeros_hbm, out_hbm,
                src_v, dst_v, *rest):
    bufs = rest[:NBUF]
    zsp = rest[NBUF]
    acc = rest[NBUF + 1]
    gsems = rest[NBUF + 2:NBUF + 2 + NBUF]
    ssems = rest[NBUF + 2 + NBUF:]
    c = lax.axis_index("c")
    s = lax.axis_index("s")
    pltpu.sync_copy(zeros_hbm.at[pl.ds(s * RPT, RPT)],
                    acc.at[pl.ds(s * RPT, RPT)])
    pltpu.sync_copy(z_hbm.at[c].at[pl.ds(s * ZRT, ZRT)],
                    zsp.at[pl.ds(s * ZRT, ZRT)])
    plsc.subcore_barrier()

    def wg(b, j):
        pltpu.make_async_copy(zsp.at[src_v.at[j]], bufs[b], gsems[b]).wait()

    def ss(b, j):
        pltpu.async_copy(bufs[b], acc.at[dst_v.at[j]], ssems[b], add=True)

    def ws(b, j):
        pltpu.make_async_copy(bufs[b], acc.at[dst_v.at[j]], ssems[b]).wait()

    for p in range(NPH):
        base = s * CPT + p * CPP
        pltpu.sync_copy(src_hbm.at[pl.ds(base, CPP)], src_v)
        pltpu.sync_copy(dst_hbm.at[pl.ds(base, CPP)], dst_v)

        for b in range(NBUF):
            pltpu.async_copy(zsp.at[src_v.at[b]], bufs[b], gsems[b])
        wg(0, 0)
        ss(0, 0)

        def body(t, _):
            jb = NBUF * t + 1
            for b in range(NBUF):
                j = jb + b
                wg((b + 1) % NBUF, j)
                ss((b + 1) % NBUF, j)
                ws(b, j - 1)
                pltpu.async_copy(zsp.at[src_v.at[j + NBUF - 1]],
                                 bufs[b], gsems[b])
            return _

        lax.fori_loop(0, (CPP - NBUF) // NBUF, body, None)
        for j in range(CPP - NBUF + 1, CPP):
            wg(j % NBUF, j)
            ss(j % NBUF, j)
            ws((j - 1) % NBUF, j - 1)
        ws((CPP - 1) % NBUF, CPP - 1)

    plsc.subcore_barrier()
    pltpu.sync_copy(acc.at[pl.ds(s * RPT, RPT)],
                    out_hbm.at[c].at[pl.ds(s * RPT, RPT)])


def _halves(ref):
    return jnp.concatenate([ref[0, :N, :], ref[1, :N, :]], axis=1)


def _split(ref, val):
    ref[0, :, :] = val[:, :HH]
    ref[1, :, :] = val[:, HH:]


def _tc_mm1(x_ref, w1_ref, u1_ref):
    u1_ref[...] = jnp.dot(x_ref[...], w1_ref[...],
                          preferred_element_type=jnp.float32)


def _tc_first(deg2_ref, u1_ref, z1_ref, dis_ref):
    degh = deg2_ref[0, :N, :] + deg2_ref[1, :N, :] + 1.0
    dish = lax.rsqrt(degh)
    dis = jnp.concatenate([dish, dish], axis=1)
    dis_ref[...] = dis
    _split(z1_ref, u1_ref[...] * dis)


def _tc_mid(dis_ref, s1_ref, z1_ref, b1_ref, w2_ref, z2_ref):
    dis = dis_ref[...]
    agg = _halves(s1_ref) + _halves(z1_ref)
    h1 = jnp.maximum(agg * dis + b1_ref[...], 0.0)
    _split(z2_ref, jnp.dot(h1, w2_ref[...],
                           preferred_element_type=jnp.float32) * dis)


def _tc_last(dis_ref, s2_ref, z2_ref, b2_ref, batch_ref, wlin_ref, blin_ref,
             out_ref):
    dis = dis_ref[...]
    agg = _halves(s2_ref) + _halves(z2_ref)
    h2 = jnp.maximum(agg * dis + b2_ref[...], 0.0)
    gid = lax.broadcasted_iota(jnp.int32, (G, N), 0)
    onehot = (batch_ref[...] == gid).astype(jnp.float32)
    sums = jnp.dot(onehot, h2, preferred_element_type=jnp.float32)
    cnt = jnp.sum(onehot, axis=1, keepdims=True)
    g = sums / jnp.maximum(cnt, 1.0)
    out_ref[...] = jnp.dot(g, wlin_ref[...],
                           preferred_element_type=jnp.float32) + blin_ref[...]


def _tc_call(body, out_shape, *args):
    return pl.pallas_call(body, out_shape=out_shape)(*args)


def _tc_first_call(deg2, u1):
    return _tc_call(_tc_first,
                    (jax.ShapeDtypeStruct((NC, N, HH), jnp.float32),
                     jax.ShapeDtypeStruct((N, H), jnp.float32)),
                    deg2, u1)


def _tc_mid_call(dis, s1, z1, b1, W2):
    return _tc_call(_tc_mid, jax.ShapeDtypeStruct((NC, N, HH), jnp.float32),
                    dis, s1, z1, b1, W2)


def kernel(x, edge_index, batch, W1, b1, W2, b2, Wlin, blin):
    src = edge_index[0]
    dst = edge_index[1]
    pad = E_PAD - E
    srcp = jnp.concatenate([src, jnp.zeros((pad,), jnp.int32)]).reshape(CH_TOT, K)
    dstp = jnp.concatenate([dst, jnp.full((pad,), N, jnp.int32)]).reshape(CH_TOT, K)

    zeros_h_hbm = jnp.zeros((NP, HH), jnp.float32)
    ones_hbm = jnp.ones((K, HH), jnp.float32)

    deg2 = _sc_degree(dstp, zeros_h_hbm, ones_hbm)
    u1 = _tc_call(_tc_mm1, jax.ShapeDtypeStruct((N, H), jnp.float32), x, W1)

    z1, dis = _tc_first_call(deg2, u1)
    s1 = _sc_scatter(srcp, dstp, z1, zeros_h_hbm)
    z2 = _tc_mid_call(dis, s1, z1, b1.reshape(1, H), W2)
    s2 = _sc_scatter(srcp, dstp, z2, zeros_h_hbm)
    out = _tc_call(_tc_last, jax.ShapeDtypeStruct((G, O), jnp.float32),
                   dis, s2, z2, b2.reshape(1, H), batch.reshape(1, N),
                   Wlin, blin.reshape(1, O))
    return out
